# baseline (speedup 1.0000x reference)
import functools

import jax
import jax.numpy as jnp
from jax import lax
from jax.experimental import pallas as pl
from jax.experimental.pallas import tpu as pltpu
from jax.experimental.pallas import tpu_sc as plsc

L = 16
NC = 2
NW = 24
ROWS = 16
D = 768
COLS = D // L


def _sc_embed_add(table, idx_w, x_w):
    mesh = plsc.VectorSubcoreMesh(core_axis_name="c", subcore_axis_name="s")

    @functools.partial(
        pl.kernel,
        mesh=mesh,
        out_type=jax.ShapeDtypeStruct((NW, ROWS, D), jnp.float32),
        scratch_types=[
            pltpu.VMEM((ROWS,), jnp.int32),
            pltpu.VMEM((ROWS, D), jnp.float32),
            pltpu.VMEM((ROWS, D), jnp.float32),
            pltpu.SemaphoreType.DMA,
        ],
    )
    def k(table_hbm, idx_hbm, x_hbm, out_hbm, idx_v, rows_v, x_v, sem):
        wid = lax.axis_index("s") * NC + lax.axis_index("c")

        @pl.when(wid < NW)
        def _():
            pltpu.sync_copy(idx_hbm.at[wid], idx_v)
            gather = pltpu.async_copy(table_hbm.at[idx_v], rows_v, sem)
            pltpu.sync_copy(x_hbm.at[wid], x_v)
            gather.wait()

            def body(j, carry):
                sl = pl.ds(j * L, L)
                for r in range(ROWS):
                    x_v[r, sl] = x_v[r, sl] + rows_v[r, sl]
                return carry

            lax.fori_loop(0, COLS, body, 0)
            pltpu.sync_copy(x_v, out_hbm.at[wid])

    return k(table, idx_w, x_w)


def kernel(x23, table, idx):
    idx_w = idx.reshape(NW, ROWS).astype(jnp.int32)
    x_w = x23.reshape(NW, ROWS, D)
    out = _sc_embed_add(table, idx_w, x_w)
    return out.reshape(1, NW * ROWS, D)


# speedup vs baseline: 1.0357x; 1.0357x over previous
import functools

import jax
import jax.numpy as jnp
from jax import lax
from jax.experimental import pallas as pl
from jax.experimental.pallas import tpu as pltpu
from jax.experimental.pallas import tpu_sc as plsc

L = 16
NC = 2
NW = 24
ROWS = 16
D = 768
COLS = D // L


def _sc_embed_add(x23, table, idx):
    mesh = plsc.VectorSubcoreMesh(core_axis_name="c", subcore_axis_name="s")

    @functools.partial(
        pl.kernel,
        mesh=mesh,
        out_type=jax.ShapeDtypeStruct((1, NW * ROWS, D), jnp.float32),
        scratch_types=[
            pltpu.VMEM((ROWS,), jnp.int32),
            pltpu.VMEM((ROWS, D), jnp.float32),
            pltpu.VMEM((ROWS, D), jnp.float32),
            pltpu.SemaphoreType.DMA,
            pltpu.SemaphoreType.DMA,
        ],
    )
    def k(x_hbm, table_hbm, idx_hbm, out_hbm, idx_v, rows_v, x_v, sem_g, sem_x):
        wid = lax.axis_index("s") * NC + lax.axis_index("c")

        @pl.when(wid < NW)
        def _():
            base = wid * ROWS
            xcopy = pltpu.async_copy(x_hbm.at[0, pl.ds(base, ROWS)], x_v, sem_x)
            pltpu.sync_copy(idx_hbm.at[0, pl.ds(base, ROWS)], idx_v)
            gather = pltpu.async_copy(table_hbm.at[idx_v], rows_v, sem_g)
            xcopy.wait()
            gather.wait()

            def body(j, carry):
                sl = pl.ds(j * L, L)
                for r in range(ROWS):
                    plsc.addupdate(x_v.at[r, sl], rows_v[r, sl])
                return carry

            lax.fori_loop(0, COLS, body, 0)
            pltpu.sync_copy(x_v, out_hbm.at[0, pl.ds(base, ROWS)])

    return k(x23, table, idx)


def kernel(x23, table, idx):
    return _sc_embed_add(x23, table, idx.astype(jnp.int32))
